# Initial kernel scaffold; baseline (speedup 1.0000x reference)
import jax
import jax.numpy as jnp
from jax.experimental import pallas as pl


def kernel(sleep_acc, sleep_hr, sleep_env, life_acc, life_hr, life_env, W_enc_acc, b_enc_acc, W_enc_hr, b_enc_hr, W_enc_env, b_enc_env, W_fuse, b_fuse, ln_gamma, ln_beta, W_gate, b_gate, W_exp, b_exp, W_h1, b_h1, W_h2, b_h2):
    raise NotImplementedError("write your pallas kernel here")


# speedup vs baseline: 2.0310x; 2.0310x over previous
import functools

import jax
import jax.numpy as jnp
from jax.experimental import pallas as pl
from jax.experimental.pallas import tpu as pltpu

_B = 16384
_BLK = 1024
_PROJ = 128
_D = 128
_NEXP = 3
_EXPAND = 128
_NTASK = 7
_OUTPAD = 8


def _dot(a, b):
    return jax.lax.dot_general(a, b, (((1,), (0,)), ((), ())),
                               preferred_element_type=jnp.float32)


def _layernorm(x):
    mu = jnp.mean(x, axis=-1, keepdims=True)
    xc = x - mu
    var = jnp.mean(xc * xc, axis=-1, keepdims=True)
    return xc * jax.lax.rsqrt(var + 1e-5)


def _fused_body(sa, sh, se, la, lh, le,
                wa, ba, wh, bh, we, be, wfa, wfh, wfe, bf,
                ln_g, ln_b, wg, bg, wexp, bexp,
                wh1a, wh1b, bh1, wh2, bh2, out):
    def branch(xa, xh, xe):
        ha = jnp.maximum(_dot(xa[...], wa[...]) + ba[...], 0.0)
        hh = jnp.maximum(_dot(xh[...], wh[...]) + bh[...], 0.0)
        he = jnp.maximum(_dot(xe[...], we[...]) + be[...], 0.0)
        f = _dot(ha, wfa[...]) + _dot(hh, wfh[...]) + _dot(he, wfe[...])
        return jnp.maximum(f + bf[...], 0.0)

    def moe(f):
        x = _layernorm(f) * ln_g[...] + ln_b[...]
        logits = _dot(x, wg[...]) + bg[...]
        m = jnp.max(logits, axis=-1, keepdims=True)
        e = jnp.exp(logits - m)
        gates = e / jnp.sum(e, axis=-1, keepdims=True)
        acc = jnp.zeros_like(x)
        for k in range(_NEXP):
            eo = jnp.maximum(_dot(x, wexp[k]) + bexp[k], 0.0)
            acc = acc + gates[:, k:k + 1] * eo
        return acc

    ms = moe(branch(sa, sh, se))
    ml = moe(branch(la, lh, le))
    h = jnp.maximum(_dot(ms, wh1a[...]) + _dot(ml, wh1b[...]) + bh1[...], 0.0)
    out[...] = _dot(h, wh2[...]) + bh2[...]


@jax.jit
def kernel(sleep_acc, sleep_hr, sleep_env, life_acc, life_hr, life_env,
           W_enc_acc, b_enc_acc, W_enc_hr, b_enc_hr, W_enc_env, b_enc_env,
           W_fuse, b_fuse, ln_gamma, ln_beta, W_gate, b_gate, W_exp, b_exp,
           W_h1, b_h1, W_h2, b_h2):
    B = sleep_acc.shape[0]
    blk = _BLK if B % _BLK == 0 else B
    grid = (B // blk,)

    wfa = W_fuse[:_PROJ]
    wfh = W_fuse[_PROJ:2 * _PROJ]
    wfe = W_fuse[2 * _PROJ:]
    ba = b_enc_acc.reshape(1, _PROJ)
    bh_ = b_enc_hr.reshape(1, _PROJ)
    be = b_enc_env.reshape(1, _PROJ)
    bf = b_fuse.reshape(1, _D)
    ln_g = ln_gamma.reshape(1, _D)
    ln_b = ln_beta.reshape(1, _D)
    wg = jnp.zeros((_D, _OUTPAD), jnp.float32).at[:, :_NEXP].set(W_gate)
    bg = jnp.full((1, _OUTPAD), -1e30, jnp.float32).at[0, :_NEXP].set(b_gate)
    bexp = b_exp.reshape(_NEXP, 1, _D)
    wh1a = W_h1[:_D]
    wh1b = W_h1[_D:]
    bh1 = b_h1.reshape(1, _EXPAND)
    wh2 = jnp.zeros((_EXPAND, _OUTPAD), jnp.float32).at[:, :_NTASK].set(W_h2)
    bh2 = jnp.zeros((1, _OUTPAD), jnp.float32).at[0, :_NTASK].set(b_h2)

    da, dh, de = W_enc_acc.shape[0], W_enc_hr.shape[0], W_enc_env.shape[0]

    def xspec(d):
        return pl.BlockSpec((blk, d), lambda i: (i, 0))

    def wspec(shape):
        nd = len(shape)
        return pl.BlockSpec(shape, lambda i: (0,) * nd)

    out = pl.pallas_call(
        _fused_body,
        grid=grid,
        in_specs=[
            xspec(da), xspec(dh), xspec(de),
            xspec(da), xspec(dh), xspec(de),
            wspec((da, _PROJ)), wspec((1, _PROJ)),
            wspec((dh, _PROJ)), wspec((1, _PROJ)),
            wspec((de, _PROJ)), wspec((1, _PROJ)),
            wspec((_PROJ, _D)), wspec((_PROJ, _D)), wspec((_PROJ, _D)),
            wspec((1, _D)), wspec((1, _D)), wspec((1, _D)),
            wspec((_D, _OUTPAD)), wspec((1, _OUTPAD)),
            wspec((_NEXP, _D, _D)), wspec((_NEXP, 1, _D)),
            wspec((_D, _EXPAND)), wspec((_D, _EXPAND)), wspec((1, _EXPAND)),
            wspec((_EXPAND, _OUTPAD)), wspec((1, _OUTPAD)),
        ],
        out_specs=pl.BlockSpec((blk, _OUTPAD), lambda i: (i, 0)),
        out_shape=jax.ShapeDtypeStruct((B, _OUTPAD), jnp.float32),
        compiler_params=pltpu.CompilerParams(
            dimension_semantics=("arbitrary",),
        ),
    )(sleep_acc, sleep_hr, sleep_env, life_acc, life_hr, life_env,
      W_enc_acc, ba, W_enc_hr, bh_, W_enc_env, be, wfa, wfh, wfe, bf,
      ln_g, ln_b, wg, bg, W_exp, bexp,
      wh1a, wh1b, bh1, wh2, bh2)
    return out[:, :_NTASK]


# speedup vs baseline: 2.2184x; 1.0922x over previous
import functools

import jax
import jax.numpy as jnp
from jax.experimental import pallas as pl
from jax.experimental.pallas import tpu as pltpu

_B = 16384
_BLK = 1024
_PROJ = 128
_D = 128
_NEXP = 3
_EXPAND = 128
_NTASK = 7
_OUTPAD = 8


def _dot(a, b):
    return jax.lax.dot_general(a, b, (((1,), (0,)), ((), ())),
                               preferred_element_type=jnp.float32)


def _layernorm(x):
    mu = jnp.mean(x, axis=-1, keepdims=True)
    xc = x - mu
    var = jnp.mean(xc * xc, axis=-1, keepdims=True)
    return xc * jax.lax.rsqrt(var + 1e-5)


def _fused_body(sa, sh, se, la, lh, le,
                wa, ba, wh, bh, we, be, wfa, wfh, wfe, bf,
                ln_g, ln_b, wg, bg, sel, wexpc, bexpc,
                wh1a, wh1b, bh1, wh2, bh2, jn, out):
    def branch(xa, xh, xe):
        ha = jnp.maximum(_dot(xa[...], wa[...]) + ba[...], 0.0)
        hh = jnp.maximum(_dot(xh[...], wh[...]) + bh[...], 0.0)
        he = jnp.maximum(_dot(xe[...], we[...]) + be[...], 0.0)
        f = _dot(ha, wfa[...]) + _dot(hh, wfh[...]) + _dot(he, wfe[...])
        return jnp.maximum(f + bf[...], 0.0)

    def moe(f):
        mu = _dot(f, jn[...])
        xc = f - mu
        var = _dot(xc * xc, jn[...])
        x = xc * jax.lax.rsqrt(var + 1e-5) * ln_g[...] + ln_b[...]
        logits = _dot(x, wg[...]) + bg[...]
        m = jnp.max(logits, axis=-1, keepdims=True)
        e = jnp.exp(logits - m)
        gates = e / jnp.sum(e, axis=-1, keepdims=True)
        g3 = _dot(gates, sel[...])
        eo = jnp.maximum(_dot(x, wexpc[...]) + bexpc[...], 0.0)
        ge = g3 * eo
        return ge[:, :_D] + ge[:, _D:2 * _D] + ge[:, 2 * _D:]

    ms = moe(branch(sa, sh, se))
    ml = moe(branch(la, lh, le))
    h = jnp.maximum(_dot(ms, wh1a[...]) + _dot(ml, wh1b[...]) + bh1[...], 0.0)
    out[...] = _dot(h, wh2[...]) + bh2[...]


@jax.jit
def kernel(sleep_acc, sleep_hr, sleep_env, life_acc, life_hr, life_env,
           W_enc_acc, b_enc_acc, W_enc_hr, b_enc_hr, W_enc_env, b_enc_env,
           W_fuse, b_fuse, ln_gamma, ln_beta, W_gate, b_gate, W_exp, b_exp,
           W_h1, b_h1, W_h2, b_h2):
    B = sleep_acc.shape[0]
    blk = _BLK if B % _BLK == 0 else B
    grid = (B // blk,)

    wfa = W_fuse[:_PROJ]
    wfh = W_fuse[_PROJ:2 * _PROJ]
    wfe = W_fuse[2 * _PROJ:]
    ba = b_enc_acc.reshape(1, _PROJ)
    bh_ = b_enc_hr.reshape(1, _PROJ)
    be = b_enc_env.reshape(1, _PROJ)
    bf = b_fuse.reshape(1, _D)
    ln_g = ln_gamma.reshape(1, _D)
    ln_b = ln_beta.reshape(1, _D)
    wg = jnp.zeros((_D, _OUTPAD), jnp.float32).at[:, :_NEXP].set(W_gate)
    bg = jnp.full((1, _OUTPAD), -1e30, jnp.float32).at[0, :_NEXP].set(b_gate)
    sel = jnp.zeros((_OUTPAD, _NEXP * _D), jnp.float32)
    for k in range(_NEXP):
        sel = sel.at[k, k * _D:(k + 1) * _D].set(1.0)
    wexpc = jnp.transpose(W_exp, (1, 0, 2)).reshape(_D, _NEXP * _D)
    bexpc = b_exp.reshape(1, _NEXP * _D)
    jn = jnp.full((_D, _D), 1.0 / _D, jnp.float32)
    wh1a = W_h1[:_D]
    wh1b = W_h1[_D:]
    bh1 = b_h1.reshape(1, _EXPAND)
    wh2 = jnp.zeros((_EXPAND, _OUTPAD), jnp.float32).at[:, :_NTASK].set(W_h2)
    bh2 = jnp.zeros((1, _OUTPAD), jnp.float32).at[0, :_NTASK].set(b_h2)

    da, dh, de = W_enc_acc.shape[0], W_enc_hr.shape[0], W_enc_env.shape[0]

    def xspec(d):
        return pl.BlockSpec((blk, d), lambda i: (i, 0))

    def wspec(shape):
        nd = len(shape)
        return pl.BlockSpec(shape, lambda i: (0,) * nd)

    out = pl.pallas_call(
        _fused_body,
        grid=grid,
        in_specs=[
            xspec(da), xspec(dh), xspec(de),
            xspec(da), xspec(dh), xspec(de),
            wspec((da, _PROJ)), wspec((1, _PROJ)),
            wspec((dh, _PROJ)), wspec((1, _PROJ)),
            wspec((de, _PROJ)), wspec((1, _PROJ)),
            wspec((_PROJ, _D)), wspec((_PROJ, _D)), wspec((_PROJ, _D)),
            wspec((1, _D)), wspec((1, _D)), wspec((1, _D)),
            wspec((_D, _OUTPAD)), wspec((1, _OUTPAD)),
            wspec((_OUTPAD, _NEXP * _D)), wspec((_D, _NEXP * _D)),
            wspec((1, _NEXP * _D)),
            wspec((_D, _EXPAND)), wspec((_D, _EXPAND)), wspec((1, _EXPAND)),
            wspec((_EXPAND, _OUTPAD)), wspec((1, _OUTPAD)),
            wspec((_D, _D)),
        ],
        out_specs=pl.BlockSpec((blk, _OUTPAD), lambda i: (i, 0)),
        out_shape=jax.ShapeDtypeStruct((B, _OUTPAD), jnp.float32),
        compiler_params=pltpu.CompilerParams(
            dimension_semantics=("arbitrary",),
        ),
    )(sleep_acc, sleep_hr, sleep_env, life_acc, life_hr, life_env,
      W_enc_acc, ba, W_enc_hr, bh_, W_enc_env, be, wfa, wfh, wfe, bf,
      ln_g, ln_b, wg, bg, sel, wexpc, bexpc,
      wh1a, wh1b, bh1, wh2, bh2, jn)
    return out[:, :_NTASK]


# speedup vs baseline: 2.4913x; 1.1230x over previous
import functools

import jax
import jax.numpy as jnp
from jax.experimental import pallas as pl
from jax.experimental.pallas import tpu as pltpu

_B = 16384
_BLK = 2048
_PROJ = 128
_D = 128
_NEXP = 3
_EXPAND = 128
_NTASK = 7
_OUTPAD = 8


def _dot(a, b):
    return jax.lax.dot_general(a, b, (((1,), (0,)), ((), ())),
                               preferred_element_type=jnp.float32)


def _layernorm(x):
    mu = jnp.mean(x, axis=-1, keepdims=True)
    xc = x - mu
    var = jnp.mean(xc * xc, axis=-1, keepdims=True)
    return xc * jax.lax.rsqrt(var + 1e-5)


def _fused_body(sa, sh, se, la, lh, le,
                wa, ba, wh, bh, we, be, wfa, wfh, wfe, bf,
                ln_g, ln_b, wg, bg, sel, wexpc, bexpc,
                wh1a, wh1b, bh1, wh2, bh2, jn, out):
    def branch(xa, xh, xe):
        ha = jnp.maximum(_dot(xa[...], wa[...]) + ba[...], 0.0)
        hh = jnp.maximum(_dot(xh[...], wh[...]) + bh[...], 0.0)
        he = jnp.maximum(_dot(xe[...], we[...]) + be[...], 0.0)
        f = _dot(ha, wfa[...]) + _dot(hh, wfh[...]) + _dot(he, wfe[...])
        return jnp.maximum(f + bf[...], 0.0)

    def moe(f):
        mu = _dot(f, jn[...])
        xc = f - mu
        var = _dot(xc * xc, jn[...])
        x = xc * jax.lax.rsqrt(var + 1e-5) * ln_g[...] + ln_b[...]
        logits = _dot(x, wg[...]) + bg[...]
        m = jnp.max(logits, axis=-1, keepdims=True)
        e = jnp.exp(logits - m)
        gates = e / jnp.sum(e, axis=-1, keepdims=True)
        g3 = _dot(gates, sel[...])
        eo = jnp.maximum(_dot(x, wexpc[...]) + bexpc[...], 0.0)
        ge = g3 * eo
        return ge[:, :_D] + ge[:, _D:2 * _D] + ge[:, 2 * _D:]

    ms = moe(branch(sa, sh, se))
    ml = moe(branch(la, lh, le))
    h = jnp.maximum(_dot(ms, wh1a[...]) + _dot(ml, wh1b[...]) + bh1[...], 0.0)
    out[...] = _dot(h, wh2[...]) + bh2[...]


@jax.jit
def kernel(sleep_acc, sleep_hr, sleep_env, life_acc, life_hr, life_env,
           W_enc_acc, b_enc_acc, W_enc_hr, b_enc_hr, W_enc_env, b_enc_env,
           W_fuse, b_fuse, ln_gamma, ln_beta, W_gate, b_gate, W_exp, b_exp,
           W_h1, b_h1, W_h2, b_h2):
    B = sleep_acc.shape[0]
    blk = _BLK if B % _BLK == 0 else B
    grid = (B // blk,)

    wfa = W_fuse[:_PROJ]
    wfh = W_fuse[_PROJ:2 * _PROJ]
    wfe = W_fuse[2 * _PROJ:]
    ba = b_enc_acc.reshape(1, _PROJ)
    bh_ = b_enc_hr.reshape(1, _PROJ)
    be = b_enc_env.reshape(1, _PROJ)
    bf = b_fuse.reshape(1, _D)
    ln_g = ln_gamma.reshape(1, _D)
    ln_b = ln_beta.reshape(1, _D)
    wg = jnp.zeros((_D, _OUTPAD), jnp.float32).at[:, :_NEXP].set(W_gate)
    bg = jnp.full((1, _OUTPAD), -1e30, jnp.float32).at[0, :_NEXP].set(b_gate)
    sel = jnp.zeros((_OUTPAD, _NEXP * _D), jnp.float32)
    for k in range(_NEXP):
        sel = sel.at[k, k * _D:(k + 1) * _D].set(1.0)
    wexpc = jnp.transpose(W_exp, (1, 0, 2)).reshape(_D, _NEXP * _D)
    bexpc = b_exp.reshape(1, _NEXP * _D)
    jn = jnp.full((_D, _D), 1.0 / _D, jnp.float32)
    wh1a = W_h1[:_D]
    wh1b = W_h1[_D:]
    bh1 = b_h1.reshape(1, _EXPAND)
    wh2 = jnp.zeros((_EXPAND, _OUTPAD), jnp.float32).at[:, :_NTASK].set(W_h2)
    bh2 = jnp.zeros((1, _OUTPAD), jnp.float32).at[0, :_NTASK].set(b_h2)

    da, dh, de = W_enc_acc.shape[0], W_enc_hr.shape[0], W_enc_env.shape[0]

    def xspec(d):
        return pl.BlockSpec((blk, d), lambda i: (i, 0))

    def wspec(shape):
        nd = len(shape)
        return pl.BlockSpec(shape, lambda i: (0,) * nd)

    out = pl.pallas_call(
        _fused_body,
        grid=grid,
        in_specs=[
            xspec(da), xspec(dh), xspec(de),
            xspec(da), xspec(dh), xspec(de),
            wspec((da, _PROJ)), wspec((1, _PROJ)),
            wspec((dh, _PROJ)), wspec((1, _PROJ)),
            wspec((de, _PROJ)), wspec((1, _PROJ)),
            wspec((_PROJ, _D)), wspec((_PROJ, _D)), wspec((_PROJ, _D)),
            wspec((1, _D)), wspec((1, _D)), wspec((1, _D)),
            wspec((_D, _OUTPAD)), wspec((1, _OUTPAD)),
            wspec((_OUTPAD, _NEXP * _D)), wspec((_D, _NEXP * _D)),
            wspec((1, _NEXP * _D)),
            wspec((_D, _EXPAND)), wspec((_D, _EXPAND)), wspec((1, _EXPAND)),
            wspec((_EXPAND, _OUTPAD)), wspec((1, _OUTPAD)),
            wspec((_D, _D)),
        ],
        out_specs=pl.BlockSpec((blk, _OUTPAD), lambda i: (i, 0)),
        out_shape=jax.ShapeDtypeStruct((B, _OUTPAD), jnp.float32),
        compiler_params=pltpu.CompilerParams(
            dimension_semantics=("arbitrary",),
        ),
    )(sleep_acc, sleep_hr, sleep_env, life_acc, life_hr, life_env,
      W_enc_acc, ba, W_enc_hr, bh_, W_enc_env, be, wfa, wfh, wfe, bf,
      ln_g, ln_b, wg, bg, sel, wexpc, bexpc,
      wh1a, wh1b, bh1, wh2, bh2, jn)
    return out[:, :_NTASK]
